# baseline (speedup 1.0000x reference)
import functools

import jax
import jax.numpy as jnp
from jax import lax
from jax.experimental import pallas as pl
from jax.experimental.pallas import tpu as pltpu
from jax.experimental.pallas import tpu_sc as plsc

_N = 50000
_E = 1600000
_W = 16
_KW = 64
_KI = 4
_K2 = _W * _W

_NC = 2
_NS = 16
_NW = _NC * _NS
_EPW = _E // _NW
_C = 2000
_NCHUNK = _EPW // _C
_RPT = _N // _NS

_BE = 6400
_BN = 5000

_EP8 = _E // 8
_G = _BE // 8
_NPAIR = _E // _G

_mesh = plsc.VectorSubcoreMesh(core_axis_name="c", subcore_axis_name="s")
_SC_PARAMS = pltpu.CompilerParams(use_tc_tiling_on_sc=False)


def _fill_rows(ref, n_rows, value):
    vec = jnp.full((_W,), value, jnp.float32)

    def body(i, carry):
        ref[i, :] = vec
        return carry

    lax.fori_loop(0, n_rows, body, 0)


@functools.partial(
    pl.kernel,
    out_type=jax.ShapeDtypeStruct((_EP8, 128), jnp.float32),
    mesh=_mesh,
    compiler_params=_SC_PARAMS,
    scratch_types=[
        pltpu.VMEM((_G,), jnp.int32),
        pltpu.VMEM((_G, _W), jnp.float32),
        pltpu.SemaphoreType.DMA,
    ],
)
def _sc_gather(h_hbm, src_hbm, xjp_hbm, idx_v, rows_v, sem):
    cid = lax.axis_index("c")
    sid = lax.axis_index("s")
    wid = sid * _NC + cid
    n_i = jnp.where(wid < _NPAIR - (_NPAIR // _NW) * _NW,
                    _NPAIR // _NW + 1, _NPAIR // _NW)

    def body(i, carry):
        p = wid + _NW * i
        b = p // 8
        j = p - 8 * b
        eoff = b * _BE + j * _G
        pltpu.sync_copy(src_hbm.at[pl.ds(eoff, _G)], idx_v)
        pltpu.async_copy(h_hbm.at[idx_v], rows_v, sem).wait()
        pltpu.sync_copy(rows_v,
                        xjp_hbm.at[pl.ds(b * _G, _G), pl.ds(j * _W, _W)])
        return carry

    lax.fori_loop(0, n_i, body, 0)


def _zero_tile_slice(rows_v, acc_sh, row0):
    pltpu.sync_copy(rows_v, acc_sh.at[pl.ds(row0, _C)])
    pltpu.sync_copy(rows_v.at[pl.ds(0, _RPT - _C)],
                    acc_sh.at[pl.ds(row0 + _C, _RPT - _C)])


@functools.partial(
    pl.kernel,
    out_type=jax.ShapeDtypeStruct((_NC, _N, _W), jnp.float32),
    mesh=_mesh,
    compiler_params=_SC_PARAMS,
    scratch_types=[
        pltpu.VMEM((_G,), jnp.int32),
        pltpu.VMEM((_C, _W), jnp.float32),
        pltpu.VMEM_SHARED((_N, _W), jnp.float32),
    ],
)
def _sc_scatter(msgp_hbm, dst_hbm, sum_hbm, idx_v, rows_v, acc_sh):
    cid = lax.axis_index("c")
    sid = lax.axis_index("s")
    wid = sid * _NC + cid
    n_i = jnp.where(wid < _NPAIR - (_NPAIR // _NW) * _NW,
                    _NPAIR // _NW + 1, _NPAIR // _NW)

    _fill_rows(rows_v, _C, 0.0)
    row0 = sid * _RPT
    _zero_tile_slice(rows_v, acc_sh, row0)
    plsc.subcore_barrier()

    def loop(i, carry):
        p = wid + _NW * i
        b = p // 8
        j = p - 8 * b
        eoff = b * _BE + j * _G
        pltpu.sync_copy(dst_hbm.at[pl.ds(eoff, _G)], idx_v)
        pltpu.sync_copy(msgp_hbm.at[pl.ds(b * _G, _G), pl.ds(j * _W, _W)],
                        rows_v.at[pl.ds(0, _G)])
        pltpu.sync_copy(rows_v.at[pl.ds(0, _G)], acc_sh.at[idx_v], add=True)
        return carry

    lax.fori_loop(0, n_i, loop, 0)
    plsc.subcore_barrier()

    pltpu.sync_copy(acc_sh.at[pl.ds(row0, _RPT)],
                    sum_hbm.at[cid, pl.ds(row0, _RPT)])


@functools.partial(
    pl.kernel,
    out_type=jax.ShapeDtypeStruct((_NC, _N, _W), jnp.float32),
    mesh=_mesh,
    compiler_params=_SC_PARAMS,
    scratch_types=[
        pltpu.VMEM((_C,), jnp.int32),
        pltpu.VMEM((_C, _W), jnp.float32),
        pltpu.VMEM_SHARED((_N, _W), jnp.float32),
    ],
)
def _sc_count(dst_hbm, cnt_hbm, idx_v, ones_v, cnt_sh):
    cid = lax.axis_index("c")
    sid = lax.axis_index("s")
    wid = sid * _NC + cid
    base = wid * _EPW

    _fill_rows(ones_v, _C, 0.0)
    row0 = sid * _RPT
    _zero_tile_slice(ones_v, cnt_sh, row0)
    plsc.subcore_barrier()
    _fill_rows(ones_v, _C, 1.0)

    def loop(i, carry):
        off = base + i * _C
        pltpu.sync_copy(dst_hbm.at[pl.ds(off, _C)], idx_v)
        pltpu.sync_copy(ones_v, cnt_sh.at[idx_v], add=True)
        return carry

    lax.fori_loop(0, _NCHUNK, loop, 0)
    plsc.subcore_barrier()

    pltpu.sync_copy(cnt_sh.at[pl.ds(row0, _RPT)],
                    cnt_hbm.at[cid, pl.ds(row0, _RPT)])


def _msg_body(ea_ref, xjp_ref, w0, b0, w1, b1, w2, b2, t_ref, s_ref,
              out_ref):
    f32 = jnp.float32
    bf16 = jnp.bfloat16
    xjp = xjp_ref[...]
    xj = jnp.concatenate(
        [xjp[:, j * _W:(j + 1) * _W] for j in range(8)], axis=0)
    eap = ea_ref[...]
    ea = jnp.concatenate(
        [eap[:, m * _KI:(m + 1) * _KI] for m in range(32)], axis=0)
    kh = jnp.dot(ea.astype(bf16), w0[...],
                 preferred_element_type=f32) + b0[...]
    kh = jnp.maximum(kh, 0.0)
    kh = jnp.dot(kh.astype(bf16), w1[...],
                 preferred_element_type=f32) + b1[...]
    kh = jnp.maximum(kh, 0.0)
    wgt = jnp.dot(kh.astype(bf16), w2[...],
                  preferred_element_type=f32) + b2[...]
    xrep = jnp.dot(xj.astype(bf16), t_ref[...], preferred_element_type=f32)
    msg = jnp.dot((wgt * xrep).astype(bf16), s_ref[...],
                  preferred_element_type=f32)
    for j in range(8):
        out_ref[:, j * _W:(j + 1) * _W] = msg[j * _G:(j + 1) * _G, :]


def _msg_call(edge_attr, xjp, w0, b0, w1, b1, w2, b2, t_m, s_m):
    bf16 = jnp.bfloat16
    full = lambda r, c: pl.BlockSpec((r, c), lambda i: (0, 0))
    return pl.pallas_call(
        _msg_body,
        grid=(_E // _BE,),
        in_specs=[
            pl.BlockSpec((_BE // 32, 128), lambda i: (i, 0)),
            pl.BlockSpec((_G, 128), lambda i: (i, 0)),
            full(_KI, _KW), full(1, _KW),
            full(_KW, _KW), full(1, _KW),
            full(_KW, _K2), full(1, _K2),
            full(_W, _K2), full(_K2, _W),
        ],
        out_specs=pl.BlockSpec((_G, 128), lambda i: (i, 0)),
        out_shape=jax.ShapeDtypeStruct((_EP8, 128), jnp.float32),
    )(edge_attr.reshape(_E // 32, 128), xjp, w0.astype(bf16),
      b0.reshape(1, _KW),
      w1.astype(bf16), b1.reshape(1, _KW), w2.astype(bf16),
      b2.reshape(1, _K2), t_m.astype(bf16), s_m.astype(bf16))


def _h0_body(x_ref, w_ref, b_ref, o_ref):
    o_ref[...] = x_ref[...] * w_ref[...] + b_ref[...]


def _h0_call(x, fc1_w, fc1_b):
    return pl.pallas_call(
        _h0_body,
        grid=(_N // _BN,),
        in_specs=[
            pl.BlockSpec((_BN, 1), lambda i: (i, 0)),
            pl.BlockSpec((1, _W), lambda i: (0, 0)),
            pl.BlockSpec((1, _W), lambda i: (0, 0)),
        ],
        out_specs=pl.BlockSpec((_BN, _W), lambda i: (i, 0)),
        out_shape=jax.ShapeDtypeStruct((_N, _W), jnp.float32),
    )(x, fc1_w, fc1_b.reshape(1, _W))


def _upd_body(final, s_ref, c_ref, h_ref, root_ref, bias_ref, f2w_ref,
              f2b_ref, o_ref):
    f32 = jnp.float32
    s = s_ref[0] + s_ref[1]
    cnt = jnp.maximum(c_ref[0] + c_ref[1], 1.0)
    hr = jnp.dot(h_ref[...], root_ref[...], preferred_element_type=f32)
    h_new = jnp.maximum(s / cnt + hr + bias_ref[...], 0.0)
    if final:
        o_ref[...] = (jnp.dot(h_new, f2w_ref[...], preferred_element_type=f32)
                      + f2b_ref[...])
    else:
        o_ref[...] = h_new


def _upd_call(final, sums, cnts, h, root, conv_bias, fc2_w, fc2_b):
    out_w = 1 if final else _W
    return pl.pallas_call(
        functools.partial(_upd_body, final),
        grid=(_N // _BN,),
        in_specs=[
            pl.BlockSpec((_NC, _BN, _W), lambda i: (0, i, 0)),
            pl.BlockSpec((_NC, _BN, _W), lambda i: (0, i, 0)),
            pl.BlockSpec((_BN, _W), lambda i: (i, 0)),
            pl.BlockSpec((_W, _W), lambda i: (0, 0)),
            pl.BlockSpec((1, _W), lambda i: (0, 0)),
            pl.BlockSpec((_W, 1), lambda i: (0, 0)),
            pl.BlockSpec((1, 1), lambda i: (0, 0)),
        ],
        out_specs=pl.BlockSpec((_BN, out_w), lambda i: (i, 0)),
        out_shape=jax.ShapeDtypeStruct((_N, out_w), jnp.float32),
    )(sums, cnts, h, root, conv_bias.reshape(1, _W), fc2_w,
      fc2_b.reshape(1, 1))


def kernel(x, edge_index, edge_attr, fc1_w, fc1_b, ker_w0, ker_b0, ker_w1,
           ker_b1, ker_w2, ker_b2, root, conv_bias, fc2_w, fc2_b):
    def _perm(a):
        return a.reshape(_E // _BE, _BE // 32, 32).transpose(0, 2, 1)\
                .reshape(_E)

    src = _perm(edge_index[0])
    dst = _perm(edge_index[1])
    eye = jnp.eye(_W, dtype=jnp.float32)
    t_m = jnp.kron(eye, jnp.ones((1, _W), jnp.float32))
    s_m = jnp.kron(jnp.ones((_W, 1), jnp.float32), eye)

    h = _h0_call(x, fc1_w, fc1_b)
    cnts = _sc_count(dst)
    for it in range(2):
        xj = _sc_gather(h, src)
        msg = _msg_call(edge_attr, xj, ker_w0, ker_b0, ker_w1, ker_b1,
                        ker_w2, ker_b2, t_m, s_m)
        sums = _sc_scatter(msg, dst)
        h = _upd_call(it == 1, sums, cnts, h, root, conv_bias, fc2_w, fc2_b)
    return h


# speedup vs baseline: 1.5667x; 1.5667x over previous
import functools

import jax
import jax.numpy as jnp
from jax import lax
from jax.experimental import pallas as pl
from jax.experimental.pallas import tpu as pltpu
from jax.experimental.pallas import tpu_sc as plsc

_N = 50000
_E = 1600000
_W = 16
_KW = 64
_KI = 4
_K2 = _W * _W

_NC = 2
_NS = 16
_NW = _NC * _NS
_EPW = _E // _NW
_C = 2000
_NCHUNK = _EPW // _C
_RPT = _N // _NS

_BE = 6400
_BN = 5000

_EP8 = _E // 8
_G = _BE // 8
_NPAIR = _E // _G

_mesh = plsc.VectorSubcoreMesh(core_axis_name="c", subcore_axis_name="s")
_SC_PARAMS = pltpu.CompilerParams(use_tc_tiling_on_sc=False)


def _fill_rows(ref, n_rows, value):
    vec = jnp.full((_W,), value, jnp.float32)

    def body(i, carry):
        ref[i, :] = vec
        return carry

    lax.fori_loop(0, n_rows, body, 0)


@functools.partial(
    pl.kernel,
    out_type=jax.ShapeDtypeStruct((_EP8, 128), jnp.float32),
    mesh=_mesh,
    compiler_params=_SC_PARAMS,
    scratch_types=[
        pltpu.VMEM((_G,), jnp.int32),
        pltpu.VMEM((_G, _W), jnp.float32),
        pltpu.SemaphoreType.DMA,
    ],
)
def _sc_gather(h_hbm, src_hbm, xjp_hbm, idx_v, rows_v, sem):
    cid = lax.axis_index("c")
    sid = lax.axis_index("s")
    wid = sid * _NC + cid
    n_i = jnp.where(wid < _NPAIR - (_NPAIR // _NW) * _NW,
                    _NPAIR // _NW + 1, _NPAIR // _NW)

    def body(i, carry):
        p = wid + _NW * i
        b = p // 8
        j = p - 8 * b
        eoff = b * _BE + j * _G
        pltpu.sync_copy(src_hbm.at[pl.ds(eoff, _G)], idx_v)
        pltpu.async_copy(h_hbm.at[idx_v], rows_v, sem).wait()
        pltpu.sync_copy(rows_v,
                        xjp_hbm.at[pl.ds(b * _G, _G), pl.ds(j * _W, _W)])
        return carry

    lax.fori_loop(0, n_i, body, 0)


def _zero_tile_slice(rows_v, acc_sh, row0):
    pltpu.sync_copy(rows_v, acc_sh.at[pl.ds(row0, _C)])
    pltpu.sync_copy(rows_v.at[pl.ds(0, _RPT - _C)],
                    acc_sh.at[pl.ds(row0 + _C, _RPT - _C)])


@functools.partial(
    pl.kernel,
    out_type=jax.ShapeDtypeStruct((_NC, _N, _W), jnp.float32),
    mesh=_mesh,
    compiler_params=_SC_PARAMS,
    scratch_types=[
        pltpu.VMEM((_G,), jnp.int32),
        pltpu.VMEM((_C, _W), jnp.float32),
        pltpu.VMEM_SHARED((_N, _W), jnp.float32),
    ],
)
def _sc_scatter(msgp_hbm, dst_hbm, sum_hbm, idx_v, rows_v, acc_sh):
    cid = lax.axis_index("c")
    sid = lax.axis_index("s")
    wid = sid * _NC + cid
    n_i = jnp.where(wid < _NPAIR - (_NPAIR // _NW) * _NW,
                    _NPAIR // _NW + 1, _NPAIR // _NW)

    _fill_rows(rows_v, _C, 0.0)
    row0 = sid * _RPT
    _zero_tile_slice(rows_v, acc_sh, row0)
    plsc.subcore_barrier()

    def loop(i, carry):
        p = wid + _NW * i
        b = p // 8
        j = p - 8 * b
        eoff = b * _BE + j * _G
        pltpu.sync_copy(dst_hbm.at[pl.ds(eoff, _G)], idx_v)
        pltpu.sync_copy(msgp_hbm.at[pl.ds(b * _G, _G), pl.ds(j * _W, _W)],
                        rows_v.at[pl.ds(0, _G)])
        pltpu.sync_copy(rows_v.at[pl.ds(0, _G)], acc_sh.at[idx_v], add=True)
        return carry

    lax.fori_loop(0, n_i, loop, 0)
    plsc.subcore_barrier()

    pltpu.sync_copy(acc_sh.at[pl.ds(row0, _RPT)],
                    sum_hbm.at[cid, pl.ds(row0, _RPT)])


@functools.partial(
    pl.kernel,
    out_type=jax.ShapeDtypeStruct((_NC, _N, _W), jnp.float32),
    mesh=_mesh,
    compiler_params=_SC_PARAMS,
    scratch_types=[
        pltpu.VMEM((_C,), jnp.int32),
        pltpu.VMEM((_C, _W), jnp.float32),
        pltpu.VMEM_SHARED((_N, _W), jnp.float32),
    ],
)
def _sc_count(dst_hbm, cnt_hbm, idx_v, ones_v, cnt_sh):
    cid = lax.axis_index("c")
    sid = lax.axis_index("s")
    wid = sid * _NC + cid
    base = wid * _EPW

    _fill_rows(ones_v, _C, 0.0)
    row0 = sid * _RPT
    _zero_tile_slice(ones_v, cnt_sh, row0)
    plsc.subcore_barrier()
    _fill_rows(ones_v, _C, 1.0)

    def loop(i, carry):
        off = base + i * _C
        pltpu.sync_copy(dst_hbm.at[pl.ds(off, _C)], idx_v)
        pltpu.sync_copy(ones_v, cnt_sh.at[idx_v], add=True)
        return carry

    lax.fori_loop(0, _NCHUNK, loop, 0)
    plsc.subcore_barrier()

    pltpu.sync_copy(cnt_sh.at[pl.ds(row0, _RPT)],
                    cnt_hbm.at[cid, pl.ds(row0, _RPT)])


def _msg_body(ea_ref, xjp_ref, w0, b0, w1, b1, w2, b2, t_ref, s_ref,
              out_ref):
    f32 = jnp.float32
    bf16 = jnp.bfloat16
    xjp = xjp_ref[...]
    xj = jnp.concatenate(
        [xjp[:, j * _W:(j + 1) * _W] for j in range(8)], axis=0)
    kh = jnp.dot(ea_ref[...].astype(bf16), w0[...],
                 preferred_element_type=f32) + b0[...]
    kh = jnp.maximum(kh, 0.0)
    kh = jnp.dot(kh.astype(bf16), w1[...],
                 preferred_element_type=f32) + b1[...]
    kh = jnp.maximum(kh, 0.0)
    wgt = jnp.dot(kh.astype(bf16), w2[...],
                  preferred_element_type=f32) + b2[...]
    xrep = jnp.dot(xj.astype(bf16), t_ref[...], preferred_element_type=f32)
    msg = jnp.dot((wgt * xrep).astype(bf16), s_ref[...],
                  preferred_element_type=f32)
    for j in range(8):
        out_ref[:, j * _W:(j + 1) * _W] = msg[j * _G:(j + 1) * _G, :]


def _msg_call(edge_attr, xjp, w0, b0, w1, b1, w2, b2, t_m, s_m):
    bf16 = jnp.bfloat16
    full = lambda r, c: pl.BlockSpec((r, c), lambda i: (0, 0))
    return pl.pallas_call(
        _msg_body,
        grid=(_E // _BE,),
        in_specs=[
            pl.BlockSpec((_BE, _KI), lambda i: (i, 0)),
            pl.BlockSpec((_G, 128), lambda i: (i, 0)),
            full(_KI, _KW), full(1, _KW),
            full(_KW, _KW), full(1, _KW),
            full(_KW, _K2), full(1, _K2),
            full(_W, _K2), full(_K2, _W),
        ],
        out_specs=pl.BlockSpec((_G, 128), lambda i: (i, 0)),
        out_shape=jax.ShapeDtypeStruct((_EP8, 128), jnp.float32),
    )(edge_attr, xjp, w0.astype(bf16),
      b0.reshape(1, _KW),
      w1.astype(bf16), b1.reshape(1, _KW), w2.astype(bf16),
      b2.reshape(1, _K2), t_m.astype(bf16), s_m.astype(bf16))


def _h0_body(x_ref, w_ref, b_ref, o_ref):
    o_ref[...] = x_ref[...] * w_ref[...] + b_ref[...]


def _h0_call(x, fc1_w, fc1_b):
    return pl.pallas_call(
        _h0_body,
        grid=(_N // _BN,),
        in_specs=[
            pl.BlockSpec((_BN, 1), lambda i: (i, 0)),
            pl.BlockSpec((1, _W), lambda i: (0, 0)),
            pl.BlockSpec((1, _W), lambda i: (0, 0)),
        ],
        out_specs=pl.BlockSpec((_BN, _W), lambda i: (i, 0)),
        out_shape=jax.ShapeDtypeStruct((_N, _W), jnp.float32),
    )(x, fc1_w, fc1_b.reshape(1, _W))


def _upd_body(final, s_ref, c_ref, h_ref, root_ref, bias_ref, f2w_ref,
              f2b_ref, o_ref):
    f32 = jnp.float32
    s = s_ref[0] + s_ref[1]
    cnt = jnp.maximum(c_ref[0] + c_ref[1], 1.0)
    hr = jnp.dot(h_ref[...], root_ref[...], preferred_element_type=f32)
    h_new = jnp.maximum(s / cnt + hr + bias_ref[...], 0.0)
    if final:
        o_ref[...] = (jnp.dot(h_new, f2w_ref[...], preferred_element_type=f32)
                      + f2b_ref[...])
    else:
        o_ref[...] = h_new


def _upd_call(final, sums, cnts, h, root, conv_bias, fc2_w, fc2_b):
    out_w = 1 if final else _W
    return pl.pallas_call(
        functools.partial(_upd_body, final),
        grid=(_N // _BN,),
        in_specs=[
            pl.BlockSpec((_NC, _BN, _W), lambda i: (0, i, 0)),
            pl.BlockSpec((_NC, _BN, _W), lambda i: (0, i, 0)),
            pl.BlockSpec((_BN, _W), lambda i: (i, 0)),
            pl.BlockSpec((_W, _W), lambda i: (0, 0)),
            pl.BlockSpec((1, _W), lambda i: (0, 0)),
            pl.BlockSpec((_W, 1), lambda i: (0, 0)),
            pl.BlockSpec((1, 1), lambda i: (0, 0)),
        ],
        out_specs=pl.BlockSpec((_BN, out_w), lambda i: (i, 0)),
        out_shape=jax.ShapeDtypeStruct((_N, out_w), jnp.float32),
    )(sums, cnts, h, root, conv_bias.reshape(1, _W), fc2_w,
      fc2_b.reshape(1, 1))


def kernel(x, edge_index, edge_attr, fc1_w, fc1_b, ker_w0, ker_b0, ker_w1,
           ker_b1, ker_w2, ker_b2, root, conv_bias, fc2_w, fc2_b):
    src = edge_index[0]
    dst = edge_index[1]
    eye = jnp.eye(_W, dtype=jnp.float32)
    t_m = jnp.kron(eye, jnp.ones((1, _W), jnp.float32))
    s_m = jnp.kron(jnp.ones((_W, 1), jnp.float32), eye)

    h = _h0_call(x, fc1_w, fc1_b)
    cnts = _sc_count(dst)
    for it in range(2):
        xj = _sc_gather(h, src)
        msg = _msg_call(edge_attr, xj, ker_w0, ker_b0, ker_w1, ker_b1,
                        ker_w2, ker_b2, t_m, s_m)
        sums = _sc_scatter(msg, dst)
        h = _upd_call(it == 1, sums, cnts, h, root, conv_bias, fc2_w, fc2_b)
    return h


# speedup vs baseline: 1.6674x; 1.0643x over previous
import functools

import jax
import jax.numpy as jnp
from jax import lax
from jax.experimental import pallas as pl
from jax.experimental.pallas import tpu as pltpu
from jax.experimental.pallas import tpu_sc as plsc

_N = 50000
_E = 1600000
_W = 16
_KW = 64
_KI = 4
_K2 = _W * _W

_NC = 2
_NS = 16
_NW = _NC * _NS
_EPW = _E // _NW
_C = 2000
_NCHUNK = _EPW // _C
_RPT = _N // _NS

_BE = 6400
_BN = 5000

_EP8 = _E // 8
_G = _BE // 8
_NPAIR = _E // _G

_mesh = plsc.VectorSubcoreMesh(core_axis_name="c", subcore_axis_name="s")
_SC_PARAMS = pltpu.CompilerParams(use_tc_tiling_on_sc=False)


def _fill_rows(ref, n_rows, value):
    vec = jnp.full((_W,), value, jnp.float32)

    def body(i, carry):
        ref[i, :] = vec
        return carry

    lax.fori_loop(0, n_rows, body, 0)


@functools.partial(
    pl.kernel,
    out_type=jax.ShapeDtypeStruct((_EP8, 128), jnp.float32),
    mesh=_mesh,
    compiler_params=_SC_PARAMS,
    scratch_types=[
        pltpu.VMEM((2, _G), jnp.int32),
        pltpu.VMEM((2, _G, _W), jnp.float32),
        pltpu.SemaphoreType.DMA((2, 3)),
    ],
)
def _sc_gather(h_hbm, src_hbm, xjp_hbm, idx_v, rows_v, sems):
    cid = lax.axis_index("c")
    sid = lax.axis_index("s")
    wid = sid * _NC + cid
    n_even = _NPAIR // _NW
    n_extra = _NPAIR - n_even * _NW

    def chunk_coords(i):
        p = wid + _NW * i
        b = p // 8
        j = p - 8 * b
        return b, j, b * _BE + j * _G

    def body(ih, carry):
        coords = [chunk_coords(2 * ih), chunk_coords(2 * ih + 1)]
        d_idx = [pltpu.async_copy(src_hbm.at[pl.ds(coords[u][2], _G)],
                                  idx_v.at[u], sems.at[u, 0])
                 for u in range(2)]
        d_g = []
        for u in range(2):
            d_idx[u].wait()
            d_g.append(pltpu.async_copy(h_hbm.at[idx_v.at[u]],
                                        rows_v.at[u], sems.at[u, 1]))
        d_w = []
        for u in range(2):
            b, j, _ = coords[u]
            d_g[u].wait()
            d_w.append(pltpu.async_copy(
                rows_v.at[u],
                xjp_hbm.at[pl.ds(b * _G, _G), pl.ds(j * _W, _W)],
                sems.at[u, 2]))
        for u in range(2):
            d_w[u].wait()
        return carry

    lax.fori_loop(0, n_even // 2, body, 0)

    @pl.when(wid < n_extra)
    def _():
        b, j, eoff = chunk_coords(n_even)
        pltpu.sync_copy(src_hbm.at[pl.ds(eoff, _G)], idx_v.at[0])
        pltpu.async_copy(h_hbm.at[idx_v.at[0]], rows_v.at[0],
                         sems.at[0, 1]).wait()
        pltpu.sync_copy(rows_v.at[0],
                        xjp_hbm.at[pl.ds(b * _G, _G), pl.ds(j * _W, _W)])


def _zero_tile_slice(rows_v, acc_sh, row0):
    pltpu.sync_copy(rows_v, acc_sh.at[pl.ds(row0, _C)])
    pltpu.sync_copy(rows_v.at[pl.ds(0, _RPT - _C)],
                    acc_sh.at[pl.ds(row0 + _C, _RPT - _C)])


@functools.partial(
    pl.kernel,
    out_type=jax.ShapeDtypeStruct((_NC, _N, _W), jnp.float32),
    mesh=_mesh,
    compiler_params=_SC_PARAMS,
    scratch_types=[
        pltpu.VMEM((2, _G), jnp.int32),
        pltpu.VMEM((2, _G, _W), jnp.float32),
        pltpu.VMEM((_C, _W), jnp.float32),
        pltpu.VMEM_SHARED((_N, _W), jnp.float32),
        pltpu.SemaphoreType.DMA((2, 3)),
    ],
)
def _sc_scatter(msgp_hbm, dst_hbm, sum_hbm, idx_v, rows_v, zero_v, acc_sh,
                sems):
    cid = lax.axis_index("c")
    sid = lax.axis_index("s")
    wid = sid * _NC + cid
    n_even = _NPAIR // _NW
    n_extra = _NPAIR - n_even * _NW

    _fill_rows(zero_v, _C, 0.0)
    row0 = sid * _RPT
    _zero_tile_slice(zero_v, acc_sh, row0)
    plsc.subcore_barrier()

    def chunk_coords(i):
        p = wid + _NW * i
        b = p // 8
        j = p - 8 * b
        return b, j, b * _BE + j * _G

    def body(ih, carry):
        coords = [chunk_coords(2 * ih), chunk_coords(2 * ih + 1)]
        d_idx = [pltpu.async_copy(dst_hbm.at[pl.ds(coords[u][2], _G)],
                                  idx_v.at[u], sems.at[u, 0])
                 for u in range(2)]
        d_msg = [pltpu.async_copy(
            msgp_hbm.at[pl.ds(coords[u][0] * _G, _G),
                        pl.ds(coords[u][1] * _W, _W)],
            rows_v.at[u], sems.at[u, 1]) for u in range(2)]
        d_add = []
        for u in range(2):
            d_idx[u].wait()
            d_msg[u].wait()
            d_add.append(pltpu.async_copy(rows_v.at[u],
                                          acc_sh.at[idx_v.at[u]],
                                          sems.at[u, 2], add=True))
        for u in range(2):
            d_add[u].wait()
        return carry

    lax.fori_loop(0, n_even // 2, body, 0)

    @pl.when(wid < n_extra)
    def _():
        b, j, eoff = chunk_coords(n_even)
        pltpu.sync_copy(dst_hbm.at[pl.ds(eoff, _G)], idx_v.at[0])
        pltpu.sync_copy(msgp_hbm.at[pl.ds(b * _G, _G), pl.ds(j * _W, _W)],
                        rows_v.at[0])
        pltpu.sync_copy(rows_v.at[0], acc_sh.at[idx_v.at[0]], add=True)

    plsc.subcore_barrier()

    pltpu.sync_copy(acc_sh.at[pl.ds(row0, _RPT)],
                    sum_hbm.at[cid, pl.ds(row0, _RPT)])


@functools.partial(
    pl.kernel,
    out_type=jax.ShapeDtypeStruct((_NC, _N, _W), jnp.float32),
    mesh=_mesh,
    compiler_params=_SC_PARAMS,
    scratch_types=[
        pltpu.VMEM((_C,), jnp.int32),
        pltpu.VMEM((_C, _W), jnp.float32),
        pltpu.VMEM_SHARED((_N, _W), jnp.float32),
    ],
)
def _sc_count(dst_hbm, cnt_hbm, idx_v, ones_v, cnt_sh):
    cid = lax.axis_index("c")
    sid = lax.axis_index("s")
    wid = sid * _NC + cid
    base = wid * _EPW

    _fill_rows(ones_v, _C, 0.0)
    row0 = sid * _RPT
    _zero_tile_slice(ones_v, cnt_sh, row0)
    plsc.subcore_barrier()
    _fill_rows(ones_v, _C, 1.0)

    def loop(i, carry):
        off = base + i * _C
        pltpu.sync_copy(dst_hbm.at[pl.ds(off, _C)], idx_v)
        pltpu.sync_copy(ones_v, cnt_sh.at[idx_v], add=True)
        return carry

    lax.fori_loop(0, _NCHUNK, loop, 0)
    plsc.subcore_barrier()

    pltpu.sync_copy(cnt_sh.at[pl.ds(row0, _RPT)],
                    cnt_hbm.at[cid, pl.ds(row0, _RPT)])


def _msg_body(ea_ref, xjp_ref, w0, b0, w1, b1, w2, b2, t_ref, s_ref,
              out_ref):
    f32 = jnp.float32
    bf16 = jnp.bfloat16
    xjp = xjp_ref[...]
    xj = jnp.concatenate(
        [xjp[:, j * _W:(j + 1) * _W] for j in range(8)], axis=0)
    kh = jnp.dot(ea_ref[...].astype(bf16), w0[...],
                 preferred_element_type=f32) + b0[...]
    kh = jnp.maximum(kh, 0.0)
    kh = jnp.dot(kh.astype(bf16), w1[...],
                 preferred_element_type=f32) + b1[...]
    kh = jnp.maximum(kh, 0.0)
    wgt = jnp.dot(kh.astype(bf16), w2[...],
                  preferred_element_type=f32) + b2[...]
    xrep = jnp.dot(xj.astype(bf16), t_ref[...], preferred_element_type=f32)
    msg = jnp.dot((wgt * xrep).astype(bf16), s_ref[...],
                  preferred_element_type=f32)
    for j in range(8):
        out_ref[:, j * _W:(j + 1) * _W] = msg[j * _G:(j + 1) * _G, :]


def _msg_call(edge_attr, xjp, w0, b0, w1, b1, w2, b2, t_m, s_m):
    bf16 = jnp.bfloat16
    full = lambda r, c: pl.BlockSpec((r, c), lambda i: (0, 0))
    return pl.pallas_call(
        _msg_body,
        grid=(_E // _BE,),
        in_specs=[
            pl.BlockSpec((_BE, _KI), lambda i: (i, 0)),
            pl.BlockSpec((_G, 128), lambda i: (i, 0)),
            full(_KI, _KW), full(1, _KW),
            full(_KW, _KW), full(1, _KW),
            full(_KW, _K2), full(1, _K2),
            full(_W, _K2), full(_K2, _W),
        ],
        out_specs=pl.BlockSpec((_G, 128), lambda i: (i, 0)),
        out_shape=jax.ShapeDtypeStruct((_EP8, 128), jnp.float32),
    )(edge_attr, xjp, w0.astype(bf16),
      b0.reshape(1, _KW),
      w1.astype(bf16), b1.reshape(1, _KW), w2.astype(bf16),
      b2.reshape(1, _K2), t_m.astype(bf16), s_m.astype(bf16))


def _h0_body(x_ref, w_ref, b_ref, o_ref):
    o_ref[...] = x_ref[...] * w_ref[...] + b_ref[...]


def _h0_call(x, fc1_w, fc1_b):
    return pl.pallas_call(
        _h0_body,
        grid=(_N // _BN,),
        in_specs=[
            pl.BlockSpec((_BN, 1), lambda i: (i, 0)),
            pl.BlockSpec((1, _W), lambda i: (0, 0)),
            pl.BlockSpec((1, _W), lambda i: (0, 0)),
        ],
        out_specs=pl.BlockSpec((_BN, _W), lambda i: (i, 0)),
        out_shape=jax.ShapeDtypeStruct((_N, _W), jnp.float32),
    )(x, fc1_w, fc1_b.reshape(1, _W))


def _upd_body(final, s_ref, c_ref, h_ref, root_ref, bias_ref, f2w_ref,
              f2b_ref, o_ref):
    f32 = jnp.float32
    s = s_ref[0] + s_ref[1]
    cnt = jnp.maximum(c_ref[0] + c_ref[1], 1.0)
    hr = jnp.dot(h_ref[...], root_ref[...], preferred_element_type=f32)
    h_new = jnp.maximum(s / cnt + hr + bias_ref[...], 0.0)
    if final:
        o_ref[...] = (jnp.dot(h_new, f2w_ref[...], preferred_element_type=f32)
                      + f2b_ref[...])
    else:
        o_ref[...] = h_new


def _upd_call(final, sums, cnts, h, root, conv_bias, fc2_w, fc2_b):
    out_w = 1 if final else _W
    return pl.pallas_call(
        functools.partial(_upd_body, final),
        grid=(_N // _BN,),
        in_specs=[
            pl.BlockSpec((_NC, _BN, _W), lambda i: (0, i, 0)),
            pl.BlockSpec((_NC, _BN, _W), lambda i: (0, i, 0)),
            pl.BlockSpec((_BN, _W), lambda i: (i, 0)),
            pl.BlockSpec((_W, _W), lambda i: (0, 0)),
            pl.BlockSpec((1, _W), lambda i: (0, 0)),
            pl.BlockSpec((_W, 1), lambda i: (0, 0)),
            pl.BlockSpec((1, 1), lambda i: (0, 0)),
        ],
        out_specs=pl.BlockSpec((_BN, out_w), lambda i: (i, 0)),
        out_shape=jax.ShapeDtypeStruct((_N, out_w), jnp.float32),
    )(sums, cnts, h, root, conv_bias.reshape(1, _W), fc2_w,
      fc2_b.reshape(1, 1))


def kernel(x, edge_index, edge_attr, fc1_w, fc1_b, ker_w0, ker_b0, ker_w1,
           ker_b1, ker_w2, ker_b2, root, conv_bias, fc2_w, fc2_b):
    src = edge_index[0]
    dst = edge_index[1]
    eye = jnp.eye(_W, dtype=jnp.float32)
    t_m = jnp.kron(eye, jnp.ones((1, _W), jnp.float32))
    s_m = jnp.kron(jnp.ones((_W, 1), jnp.float32), eye)

    h = _h0_call(x, fc1_w, fc1_b)
    cnts = _sc_count(dst)
    for it in range(2):
        xj = _sc_gather(h, src)
        msg = _msg_call(edge_attr, xj, ker_w0, ker_b0, ker_w1, ker_b1,
                        ker_w2, ker_b2, t_m, s_m)
        sums = _sc_scatter(msg, dst)
        h = _upd_call(it == 1, sums, cnts, h, root, conv_bias, fc2_w, fc2_b)
    return h


# speedup vs baseline: 1.8361x; 1.1011x over previous
import functools

import jax
import jax.numpy as jnp
from jax import lax
from jax.experimental import pallas as pl
from jax.experimental.pallas import tpu as pltpu
from jax.experimental.pallas import tpu_sc as plsc

_N = 50000
_E = 1600000
_W = 16
_KW = 64
_KI = 4
_K2 = _W * _W

_NC = 2
_NS = 16
_NW = _NC * _NS
_EPW = _E // _NW
_C = 2000
_NCHUNK = _EPW // _C
_RPT = _N // _NS

_BE = 6400
_BN = 5000

_EP8 = _E // 8
_G = _BE // 8
_NPAIR = _E // _G

_mesh = plsc.VectorSubcoreMesh(core_axis_name="c", subcore_axis_name="s")
_SC_PARAMS = pltpu.CompilerParams(use_tc_tiling_on_sc=False)


def _fill_rows(ref, n_rows, value):
    vec = jnp.full((_W,), value, jnp.float32)

    def body(i, carry):
        ref[i, :] = vec
        return carry

    lax.fori_loop(0, n_rows, body, 0)


@functools.partial(
    pl.kernel,
    out_type=jax.ShapeDtypeStruct((_EP8, 128), jnp.float32),
    mesh=_mesh,
    compiler_params=_SC_PARAMS,
    scratch_types=[
        pltpu.VMEM((2, _G), jnp.int32),
        pltpu.VMEM((2, _G, _W), jnp.float32),
        pltpu.SemaphoreType.DMA((2, 3)),
    ],
)
def _sc_gather(h_hbm, src_hbm, xjp_hbm, idx_v, rows_v, sems):
    cid = lax.axis_index("c")
    sid = lax.axis_index("s")
    wid = sid * _NC + cid
    n_even = _NPAIR // _NW
    n_extra = _NPAIR - n_even * _NW

    def chunk_coords(i):
        p = wid + _NW * i
        b = p // 8
        j = p - 8 * b
        return b, j, b * _BE + j * _G

    def body(ih, carry):
        coords = [chunk_coords(2 * ih), chunk_coords(2 * ih + 1)]
        d_idx = [pltpu.async_copy(src_hbm.at[pl.ds(coords[u][2], _G)],
                                  idx_v.at[u], sems.at[u, 0])
                 for u in range(2)]
        d_g = []
        for u in range(2):
            d_idx[u].wait()
            d_g.append(pltpu.async_copy(h_hbm.at[idx_v.at[u]],
                                        rows_v.at[u], sems.at[u, 1]))
        d_w = []
        for u in range(2):
            b, j, _ = coords[u]
            d_g[u].wait()
            d_w.append(pltpu.async_copy(
                rows_v.at[u],
                xjp_hbm.at[pl.ds(b * _G, _G), pl.ds(j * _W, _W)],
                sems.at[u, 2]))
        for u in range(2):
            d_w[u].wait()
        return carry

    lax.fori_loop(0, n_even // 2, body, 0)

    @pl.when(wid < n_extra)
    def _():
        b, j, eoff = chunk_coords(n_even)
        pltpu.sync_copy(src_hbm.at[pl.ds(eoff, _G)], idx_v.at[0])
        pltpu.async_copy(h_hbm.at[idx_v.at[0]], rows_v.at[0],
                         sems.at[0, 1]).wait()
        pltpu.sync_copy(rows_v.at[0],
                        xjp_hbm.at[pl.ds(b * _G, _G), pl.ds(j * _W, _W)])


def _zero_tile_slice(rows_v, acc_sh, row0):
    pltpu.sync_copy(rows_v, acc_sh.at[pl.ds(row0, _C)])
    pltpu.sync_copy(rows_v.at[pl.ds(0, _RPT - _C)],
                    acc_sh.at[pl.ds(row0 + _C, _RPT - _C)])


@functools.partial(
    pl.kernel,
    out_type=jax.ShapeDtypeStruct((_NC, _N, _W), jnp.float32),
    mesh=_mesh,
    compiler_params=_SC_PARAMS,
    scratch_types=[
        pltpu.VMEM((2, _G), jnp.int32),
        pltpu.VMEM((2, _G, _W), jnp.float32),
        pltpu.VMEM((_C, _W), jnp.float32),
        pltpu.VMEM_SHARED((_N, _W), jnp.float32),
        pltpu.SemaphoreType.DMA((2, 3)),
    ],
)
def _sc_scatter(msgp_hbm, dst_hbm, sum_hbm, idx_v, rows_v, zero_v, acc_sh,
                sems):
    cid = lax.axis_index("c")
    sid = lax.axis_index("s")
    wid = sid * _NC + cid
    n_even = _NPAIR // _NW
    n_extra = _NPAIR - n_even * _NW

    _fill_rows(zero_v, _C, 0.0)
    row0 = sid * _RPT
    _zero_tile_slice(zero_v, acc_sh, row0)
    plsc.subcore_barrier()

    def chunk_coords(i):
        p = wid + _NW * i
        b = p // 8
        j = p - 8 * b
        return b, j, b * _BE + j * _G

    def body(ih, carry):
        coords = [chunk_coords(2 * ih), chunk_coords(2 * ih + 1)]
        d_idx = [pltpu.async_copy(dst_hbm.at[pl.ds(coords[u][2], _G)],
                                  idx_v.at[u], sems.at[u, 0])
                 for u in range(2)]
        d_msg = [pltpu.async_copy(
            msgp_hbm.at[pl.ds(coords[u][0] * _G, _G),
                        pl.ds(coords[u][1] * _W, _W)],
            rows_v.at[u], sems.at[u, 1]) for u in range(2)]
        d_add = []
        for u in range(2):
            d_idx[u].wait()
            d_msg[u].wait()
            d_add.append(pltpu.async_copy(rows_v.at[u],
                                          acc_sh.at[idx_v.at[u]],
                                          sems.at[u, 2], add=True))
        for u in range(2):
            d_add[u].wait()
        return carry

    lax.fori_loop(0, n_even // 2, body, 0)

    @pl.when(wid < n_extra)
    def _():
        b, j, eoff = chunk_coords(n_even)
        pltpu.sync_copy(dst_hbm.at[pl.ds(eoff, _G)], idx_v.at[0])
        pltpu.sync_copy(msgp_hbm.at[pl.ds(b * _G, _G), pl.ds(j * _W, _W)],
                        rows_v.at[0])
        pltpu.sync_copy(rows_v.at[0], acc_sh.at[idx_v.at[0]], add=True)

    plsc.subcore_barrier()

    pltpu.sync_copy(acc_sh.at[pl.ds(row0, _RPT)],
                    sum_hbm.at[cid, pl.ds(row0, _RPT)])


@functools.partial(
    pl.kernel,
    out_type=jax.ShapeDtypeStruct((_NC, _N, _W), jnp.float32),
    mesh=_mesh,
    compiler_params=_SC_PARAMS,
    scratch_types=[
        pltpu.VMEM((_C,), jnp.int32),
        pltpu.VMEM((_C, _W), jnp.float32),
        pltpu.VMEM_SHARED((_N, _W), jnp.float32),
    ],
)
def _sc_count(dst_hbm, cnt_hbm, idx_v, ones_v, cnt_sh):
    cid = lax.axis_index("c")
    sid = lax.axis_index("s")
    wid = sid * _NC + cid
    base = wid * _EPW

    _fill_rows(ones_v, _C, 0.0)
    row0 = sid * _RPT
    _zero_tile_slice(ones_v, cnt_sh, row0)
    plsc.subcore_barrier()
    _fill_rows(ones_v, _C, 1.0)

    def loop(i, carry):
        off = base + i * _C
        pltpu.sync_copy(dst_hbm.at[pl.ds(off, _C)], idx_v)
        pltpu.sync_copy(ones_v, cnt_sh.at[idx_v], add=True)
        return carry

    lax.fori_loop(0, _NCHUNK, loop, 0)
    plsc.subcore_barrier()

    pltpu.sync_copy(cnt_sh.at[pl.ds(row0, _RPT)],
                    cnt_hbm.at[cid, pl.ds(row0, _RPT)])


def _msg_body(ea_ref, xjp_ref, w0, b0, w1, b1, w2, b2, t_ref, s_ref,
              out_ref):
    f32 = jnp.float32
    bf16 = jnp.bfloat16
    xjp = xjp_ref[...]
    xj = jnp.concatenate(
        [xjp[:, j * _W:(j + 1) * _W] for j in range(8)], axis=0)
    kh = lax.dot_general(ea_ref[...].astype(bf16), w0[...],
                         (((0,), (0,)), ((), ())),
                         preferred_element_type=f32) + b0[...]
    kh = jnp.maximum(kh, 0.0)
    kh = jnp.dot(kh.astype(bf16), w1[...],
                 preferred_element_type=f32) + b1[...]
    kh = jnp.maximum(kh, 0.0)
    wgt = jnp.dot(kh.astype(bf16), w2[...],
                  preferred_element_type=f32) + b2[...]
    xrep = jnp.dot(xj.astype(bf16), t_ref[...], preferred_element_type=f32)
    msg = jnp.dot((wgt * xrep).astype(bf16), s_ref[...],
                  preferred_element_type=f32)
    for j in range(8):
        out_ref[:, j * _W:(j + 1) * _W] = msg[j * _G:(j + 1) * _G, :]


def _msg_call(edge_attr, xjp, w0, b0, w1, b1, w2, b2, t_m, s_m):
    bf16 = jnp.bfloat16
    full = lambda r, c: pl.BlockSpec((r, c), lambda i: (0, 0))
    return pl.pallas_call(
        _msg_body,
        grid=(_E // _BE,),
        in_specs=[
            pl.BlockSpec((_KI, _BE), lambda i: (0, i)),
            pl.BlockSpec((_G, 128), lambda i: (i, 0)),
            full(_KI, _KW), full(1, _KW),
            full(_KW, _KW), full(1, _KW),
            full(_KW, _K2), full(1, _K2),
            full(_W, _K2), full(_K2, _W),
        ],
        out_specs=pl.BlockSpec((_G, 128), lambda i: (i, 0)),
        out_shape=jax.ShapeDtypeStruct((_EP8, 128), jnp.float32),
    )(edge_attr.T, xjp, w0.astype(bf16),
      b0.reshape(1, _KW),
      w1.astype(bf16), b1.reshape(1, _KW), w2.astype(bf16),
      b2.reshape(1, _K2), t_m.astype(bf16), s_m.astype(bf16))


def _h0_body(x_ref, w_ref, b_ref, o_ref):
    o_ref[...] = x_ref[...] * w_ref[...] + b_ref[...]


def _h0_call(x, fc1_w, fc1_b):
    return pl.pallas_call(
        _h0_body,
        grid=(_N // _BN,),
        in_specs=[
            pl.BlockSpec((_BN, 1), lambda i: (i, 0)),
            pl.BlockSpec((1, _W), lambda i: (0, 0)),
            pl.BlockSpec((1, _W), lambda i: (0, 0)),
        ],
        out_specs=pl.BlockSpec((_BN, _W), lambda i: (i, 0)),
        out_shape=jax.ShapeDtypeStruct((_N, _W), jnp.float32),
    )(x, fc1_w, fc1_b.reshape(1, _W))


def _upd_body(final, s_ref, c_ref, h_ref, root_ref, bias_ref, f2w_ref,
              f2b_ref, o_ref):
    f32 = jnp.float32
    s = s_ref[0] + s_ref[1]
    cnt = jnp.maximum(c_ref[0] + c_ref[1], 1.0)
    hr = jnp.dot(h_ref[...], root_ref[...], preferred_element_type=f32)
    h_new = jnp.maximum(s / cnt + hr + bias_ref[...], 0.0)
    if final:
        o_ref[...] = (jnp.dot(h_new, f2w_ref[...], preferred_element_type=f32)
                      + f2b_ref[...])
    else:
        o_ref[...] = h_new


def _upd_call(final, sums, cnts, h, root, conv_bias, fc2_w, fc2_b):
    out_w = 1 if final else _W
    return pl.pallas_call(
        functools.partial(_upd_body, final),
        grid=(_N // _BN,),
        in_specs=[
            pl.BlockSpec((_NC, _BN, _W), lambda i: (0, i, 0)),
            pl.BlockSpec((_NC, _BN, _W), lambda i: (0, i, 0)),
            pl.BlockSpec((_BN, _W), lambda i: (i, 0)),
            pl.BlockSpec((_W, _W), lambda i: (0, 0)),
            pl.BlockSpec((1, _W), lambda i: (0, 0)),
            pl.BlockSpec((_W, 1), lambda i: (0, 0)),
            pl.BlockSpec((1, 1), lambda i: (0, 0)),
        ],
        out_specs=pl.BlockSpec((_BN, out_w), lambda i: (i, 0)),
        out_shape=jax.ShapeDtypeStruct((_N, out_w), jnp.float32),
    )(sums, cnts, h, root, conv_bias.reshape(1, _W), fc2_w,
      fc2_b.reshape(1, 1))


def kernel(x, edge_index, edge_attr, fc1_w, fc1_b, ker_w0, ker_b0, ker_w1,
           ker_b1, ker_w2, ker_b2, root, conv_bias, fc2_w, fc2_b):
    src = edge_index[0]
    dst = edge_index[1]
    eye = jnp.eye(_W, dtype=jnp.float32)
    t_m = jnp.kron(eye, jnp.ones((1, _W), jnp.float32))
    s_m = jnp.kron(jnp.ones((_W, 1), jnp.float32), eye)

    h = _h0_call(x, fc1_w, fc1_b)
    cnts = _sc_count(dst)
    for it in range(2):
        xj = _sc_gather(h, src)
        msg = _msg_call(edge_attr, xj, ker_w0, ker_b0, ker_w1, ker_b1,
                        ker_w2, ker_b2, t_m, s_m)
        sums = _sc_scatter(msg, dst)
        h = _upd_call(it == 1, sums, cnts, h, root, conv_bias, fc2_w, fc2_b)
    return h


# speedup vs baseline: 1.9822x; 1.0796x over previous
import functools

import jax
import jax.numpy as jnp
from jax import lax
from jax.experimental import pallas as pl
from jax.experimental.pallas import tpu as pltpu
from jax.experimental.pallas import tpu_sc as plsc

_N = 50000
_E = 1600000
_W = 16
_KW = 64
_KI = 4
_K2 = _W * _W

_NC = 2
_NS = 16
_NW = _NC * _NS
_EPW = _E // _NW
_C = 2000
_NCHUNK = _EPW // _C
_RPT = _N // _NS

_BE = 6400
_BN = 5000

_EP8 = _E // 8
_G = _BE // 8
_NPAIR = _E // _G

_mesh = plsc.VectorSubcoreMesh(core_axis_name="c", subcore_axis_name="s")
_SC_PARAMS = pltpu.CompilerParams(use_tc_tiling_on_sc=False)


def _fill_rows(ref, n_rows, value):
    vec = jnp.full((_W,), value, jnp.float32)

    def body(i, carry):
        ref[i, :] = vec
        return carry

    lax.fori_loop(0, n_rows, body, 0)


@functools.partial(
    pl.kernel,
    out_type=jax.ShapeDtypeStruct((_EP8, 128), jnp.float32),
    mesh=_mesh,
    compiler_params=_SC_PARAMS,
    scratch_types=[
        pltpu.VMEM((2, _G), jnp.int32),
        pltpu.VMEM((2, _G, _W), jnp.float32),
        pltpu.SemaphoreType.DMA((2, 3)),
    ],
)
def _sc_gather(h_hbm, src_hbm, xjp_hbm, idx_v, rows_v, sems):
    cid = lax.axis_index("c")
    sid = lax.axis_index("s")
    wid = sid * _NC + cid
    n_even = _NPAIR // _NW
    n_extra = _NPAIR - n_even * _NW

    def chunk_coords(i):
        p = wid + _NW * i
        b = p // 8
        j = p - 8 * b
        return b, j, b * _BE + j * _G

    def body(ih, carry):
        coords = [chunk_coords(2 * ih), chunk_coords(2 * ih + 1)]
        d_idx = [pltpu.async_copy(src_hbm.at[pl.ds(coords[u][2], _G)],
                                  idx_v.at[u], sems.at[u, 0])
                 for u in range(2)]
        d_g = []
        for u in range(2):
            d_idx[u].wait()
            d_g.append(pltpu.async_copy(h_hbm.at[idx_v.at[u]],
                                        rows_v.at[u], sems.at[u, 1]))
        d_w = []
        for u in range(2):
            b, j, _ = coords[u]
            d_g[u].wait()
            d_w.append(pltpu.async_copy(
                rows_v.at[u],
                xjp_hbm.at[pl.ds(b * _G, _G), pl.ds(j * _W, _W)],
                sems.at[u, 2]))
        for u in range(2):
            d_w[u].wait()
        return carry

    lax.fori_loop(0, n_even // 2, body, 0)

    @pl.when(wid < n_extra)
    def _():
        b, j, eoff = chunk_coords(n_even)
        pltpu.sync_copy(src_hbm.at[pl.ds(eoff, _G)], idx_v.at[0])
        pltpu.async_copy(h_hbm.at[idx_v.at[0]], rows_v.at[0],
                         sems.at[0, 1]).wait()
        pltpu.sync_copy(rows_v.at[0],
                        xjp_hbm.at[pl.ds(b * _G, _G), pl.ds(j * _W, _W)])


def _zero_tile_slice(rows_v, acc_sh, row0):
    pltpu.sync_copy(rows_v, acc_sh.at[pl.ds(row0, _C)])
    pltpu.sync_copy(rows_v.at[pl.ds(0, _RPT - _C)],
                    acc_sh.at[pl.ds(row0 + _C, _RPT - _C)])


@functools.partial(
    pl.kernel,
    out_type=jax.ShapeDtypeStruct((_NC, _N, _W), jnp.float32),
    mesh=_mesh,
    compiler_params=_SC_PARAMS,
    scratch_types=[
        pltpu.VMEM((2, _G), jnp.int32),
        pltpu.VMEM((2, _G, _W), jnp.float32),
        pltpu.VMEM((_C, _W), jnp.float32),
        pltpu.VMEM_SHARED((_N, _W), jnp.float32),
        pltpu.SemaphoreType.DMA((2, 3)),
    ],
)
def _sc_scatter(msgp_hbm, dst_hbm, sum_hbm, idx_v, rows_v, zero_v, acc_sh,
                sems):
    cid = lax.axis_index("c")
    sid = lax.axis_index("s")
    wid = sid * _NC + cid
    n_even = _NPAIR // _NW
    n_extra = _NPAIR - n_even * _NW

    _fill_rows(zero_v, _C, 0.0)
    row0 = sid * _RPT
    _zero_tile_slice(zero_v, acc_sh, row0)
    plsc.subcore_barrier()

    def chunk_coords(i):
        p = wid + _NW * i
        b = p // 8
        j = p - 8 * b
        return b, j, b * _BE + j * _G

    def body(ih, carry):
        coords = [chunk_coords(2 * ih), chunk_coords(2 * ih + 1)]
        d_idx = [pltpu.async_copy(dst_hbm.at[pl.ds(coords[u][2], _G)],
                                  idx_v.at[u], sems.at[u, 0])
                 for u in range(2)]
        d_msg = [pltpu.async_copy(
            msgp_hbm.at[pl.ds(coords[u][0] * _G, _G),
                        pl.ds(coords[u][1] * _W, _W)],
            rows_v.at[u], sems.at[u, 1]) for u in range(2)]
        d_add = []
        for u in range(2):
            d_idx[u].wait()
            d_msg[u].wait()
            d_add.append(pltpu.async_copy(rows_v.at[u],
                                          acc_sh.at[idx_v.at[u]],
                                          sems.at[u, 2], add=True))
        for u in range(2):
            d_add[u].wait()
        return carry

    lax.fori_loop(0, n_even // 2, body, 0)

    @pl.when(wid < n_extra)
    def _():
        b, j, eoff = chunk_coords(n_even)
        pltpu.sync_copy(dst_hbm.at[pl.ds(eoff, _G)], idx_v.at[0])
        pltpu.sync_copy(msgp_hbm.at[pl.ds(b * _G, _G), pl.ds(j * _W, _W)],
                        rows_v.at[0])
        pltpu.sync_copy(rows_v.at[0], acc_sh.at[idx_v.at[0]], add=True)

    plsc.subcore_barrier()

    pltpu.sync_copy(acc_sh.at[pl.ds(row0, _RPT)],
                    sum_hbm.at[cid, pl.ds(row0, _RPT)])


@functools.partial(
    pl.kernel,
    out_type=jax.ShapeDtypeStruct((_NC, _N, _W), jnp.float32),
    mesh=_mesh,
    compiler_params=_SC_PARAMS,
    scratch_types=[
        pltpu.VMEM((_C,), jnp.int32),
        pltpu.VMEM((_C, _W), jnp.float32),
        pltpu.VMEM_SHARED((_N, _W), jnp.float32),
    ],
)
def _sc_count(dst_hbm, cnt_hbm, idx_v, ones_v, cnt_sh):
    cid = lax.axis_index("c")
    sid = lax.axis_index("s")
    wid = sid * _NC + cid
    base = wid * _EPW

    _fill_rows(ones_v, _C, 0.0)
    row0 = sid * _RPT
    _zero_tile_slice(ones_v, cnt_sh, row0)
    plsc.subcore_barrier()
    _fill_rows(ones_v, _C, 1.0)

    def loop(i, carry):
        off = base + i * _C
        pltpu.sync_copy(dst_hbm.at[pl.ds(off, _C)], idx_v)
        pltpu.sync_copy(ones_v, cnt_sh.at[idx_v], add=True)
        return carry

    lax.fori_loop(0, _NCHUNK, loop, 0)
    plsc.subcore_barrier()

    pltpu.sync_copy(cnt_sh.at[pl.ds(row0, _RPT)],
                    cnt_hbm.at[cid, pl.ds(row0, _RPT)])


_BH = _BE // 2


def _msg_body(ea_ref, xjp_ref, w0, b0, w1, b1, w2, b2, t_ref, s_ref,
              out_ref):
    f32 = jnp.float32
    bf16 = jnp.bfloat16
    xjp = xjp_ref[...]
    g1 = jnp.concatenate(
        [xjp[:, j * _W:(j + 1) * _W] for j in range(4)], axis=0)
    g2 = jnp.concatenate(
        [xjp[:, j * _W:(j + 1) * _W] for j in range(4, 8)], axis=0)
    xj2 = jnp.concatenate([g1, g2], axis=1)
    ea_t = ea_ref[...]
    ea2 = jnp.concatenate([ea_t[:, :_BH], ea_t[:, _BH:]], axis=0)
    kh = lax.dot_general(ea2.astype(bf16), w0[...],
                         (((0,), (0,)), ((), ())),
                         preferred_element_type=f32) + b0[...]
    kh = jnp.maximum(kh, 0.0)
    kh = jnp.dot(kh.astype(bf16), w1[...],
                 preferred_element_type=f32) + b1[...]
    kh = jnp.maximum(kh, 0.0)
    wgt = jnp.dot(kh.astype(bf16), w2[...],
                  preferred_element_type=f32) + b2[...]
    xrep = jnp.dot(xj2.astype(bf16), t_ref[...], preferred_element_type=f32)
    msg = jnp.dot((wgt * xrep).astype(bf16), s_ref[...],
                  preferred_element_type=f32)
    for j in range(4):
        out_ref[:, j * _W:(j + 1) * _W] = \
            msg[j * _G:(j + 1) * _G, :_W]
    for j in range(4, 8):
        out_ref[:, j * _W:(j + 1) * _W] = \
            msg[(j - 4) * _G:(j - 3) * _G, _W:]


def _blockdiag2(a):
    r, c = a.shape
    z = jnp.zeros((r, c), a.dtype)
    return jnp.concatenate(
        [jnp.concatenate([a, z], axis=1),
         jnp.concatenate([z, a], axis=1)], axis=0)


def _msg_call(edge_attr, xjp, w0, b0, w1, b1, w2, b2, t_m, s_m):
    bf16 = jnp.bfloat16
    full = lambda r, c: pl.BlockSpec((r, c), lambda i: (0, 0))
    dup = lambda b: jnp.concatenate([b, b]).reshape(1, -1)
    return pl.pallas_call(
        _msg_body,
        grid=(_E // _BE,),
        in_specs=[
            pl.BlockSpec((_KI, _BE), lambda i: (0, i)),
            pl.BlockSpec((_G, 128), lambda i: (i, 0)),
            full(2 * _KI, 2 * _KW), full(1, 2 * _KW),
            full(2 * _KW, 2 * _KW), full(1, 2 * _KW),
            full(2 * _KW, 2 * _K2), full(1, 2 * _K2),
            full(2 * _W, 2 * _K2), full(2 * _K2, 2 * _W),
        ],
        out_specs=pl.BlockSpec((_G, 128), lambda i: (i, 0)),
        out_shape=jax.ShapeDtypeStruct((_EP8, 128), jnp.float32),
    )(edge_attr.T, xjp, _blockdiag2(w0).astype(bf16), dup(b0),
      _blockdiag2(w1).astype(bf16), dup(b1),
      _blockdiag2(w2).astype(bf16), dup(b2),
      _blockdiag2(t_m).astype(bf16), _blockdiag2(s_m).astype(bf16))


def _h0_body(x_ref, w_ref, b_ref, o_ref):
    o_ref[...] = x_ref[...] * w_ref[...] + b_ref[...]


def _h0_call(x, fc1_w, fc1_b):
    return pl.pallas_call(
        _h0_body,
        grid=(_N // _BN,),
        in_specs=[
            pl.BlockSpec((_BN, 1), lambda i: (i, 0)),
            pl.BlockSpec((1, _W), lambda i: (0, 0)),
            pl.BlockSpec((1, _W), lambda i: (0, 0)),
        ],
        out_specs=pl.BlockSpec((_BN, _W), lambda i: (i, 0)),
        out_shape=jax.ShapeDtypeStruct((_N, _W), jnp.float32),
    )(x, fc1_w, fc1_b.reshape(1, _W))


def _upd_body(final, s_ref, c_ref, h_ref, root_ref, bias_ref, f2w_ref,
              f2b_ref, o_ref):
    f32 = jnp.float32
    s = s_ref[0] + s_ref[1]
    cnt = jnp.maximum(c_ref[0] + c_ref[1], 1.0)
    hr = jnp.dot(h_ref[...], root_ref[...], preferred_element_type=f32)
    h_new = jnp.maximum(s / cnt + hr + bias_ref[...], 0.0)
    if final:
        o_ref[...] = (jnp.dot(h_new, f2w_ref[...], preferred_element_type=f32)
                      + f2b_ref[...])
    else:
        o_ref[...] = h_new


def _upd_call(final, sums, cnts, h, root, conv_bias, fc2_w, fc2_b):
    out_w = 1 if final else _W
    return pl.pallas_call(
        functools.partial(_upd_body, final),
        grid=(_N // _BN,),
        in_specs=[
            pl.BlockSpec((_NC, _BN, _W), lambda i: (0, i, 0)),
            pl.BlockSpec((_NC, _BN, _W), lambda i: (0, i, 0)),
            pl.BlockSpec((_BN, _W), lambda i: (i, 0)),
            pl.BlockSpec((_W, _W), lambda i: (0, 0)),
            pl.BlockSpec((1, _W), lambda i: (0, 0)),
            pl.BlockSpec((_W, 1), lambda i: (0, 0)),
            pl.BlockSpec((1, 1), lambda i: (0, 0)),
        ],
        out_specs=pl.BlockSpec((_BN, out_w), lambda i: (i, 0)),
        out_shape=jax.ShapeDtypeStruct((_N, out_w), jnp.float32),
    )(sums, cnts, h, root, conv_bias.reshape(1, _W), fc2_w,
      fc2_b.reshape(1, 1))


def kernel(x, edge_index, edge_attr, fc1_w, fc1_b, ker_w0, ker_b0, ker_w1,
           ker_b1, ker_w2, ker_b2, root, conv_bias, fc2_w, fc2_b):
    src = edge_index[0]
    dst = edge_index[1]
    eye = jnp.eye(_W, dtype=jnp.float32)
    t_m = jnp.kron(eye, jnp.ones((1, _W), jnp.float32))
    s_m = jnp.kron(jnp.ones((_W, 1), jnp.float32), eye)

    h = _h0_call(x, fc1_w, fc1_b)
    cnts = _sc_count(dst)
    for it in range(2):
        xj = _sc_gather(h, src)
        msg = _msg_call(edge_attr, xj, ker_w0, ker_b0, ker_w1, ker_b1,
                        ker_w2, ker_b2, t_m, s_m)
        sums = _sc_scatter(msg, dst)
        h = _upd_call(it == 1, sums, cnts, h, root, conv_bias, fc2_w, fc2_b)
    return h
